# baseline (speedup 1.0000x reference)
import dataclasses
import functools

import jax
import jax.numpy as jnp
from jax import lax
from jax.experimental import pallas as pl
from jax.experimental.pallas import tpu as pltpu
from jax.experimental.pallas import tpu_sc as plsc

N, D, H, E, K = 2048, 1024, 1024, 8, 2
B = 256
NBMAX = N * K // B + E - 1
P = NBMAX * B
NC, NS = 2, 16
NW = NC * NS
PAIRS_W = N * K // NW
TOK_W = N // NW

_mesh = plsc.VectorSubcoreMesh(core_axis_name="c", subcore_axis_name="s")
_cp = pltpu.CompilerParams()
if "needs_layout_passes" in pltpu.CompilerParams.__dataclass_fields__:
    _cp = dataclasses.replace(_cp, needs_layout_passes=False)


def _lane_cumsum(x):
    r, c = x.shape
    lane = jax.lax.broadcasted_iota(jnp.int32, (r, c), 1)
    acc = x
    sh = 1
    while sh < c:
        rolled = pltpu.roll(acc, sh, axis=1)
        acc = acc + jnp.where(lane >= sh, rolled, 0)
        sh *= 2
    return acc


def _routing_kernel(x_ref, gW1_ref, gb1_ref, gW2_ref, gb2_ref,
                    slots_ref, gates_ref, meta_ref):
    x = x_ref[...]
    l1 = jnp.tanh(
        jax.lax.dot_general(x, gW1_ref[...], (((1,), (0,)), ((), ())),
                            preferred_element_type=jnp.float32)
        + gb1_ref[...].reshape(1, E))
    logits = jax.lax.dot_general(l1, gW2_ref[...], (((1,), (0,)), ((), ())),
                                 preferred_element_type=jnp.float32) \
        + gb2_ref[...].reshape(1, E)
    mx = jnp.max(logits, axis=-1, keepdims=True)
    exn = jnp.exp(logits - mx)
    probs = exn / jnp.sum(exn, axis=-1, keepdims=True)
    probsT = jnp.transpose(probs)

    subl = jax.lax.broadcasted_iota(jnp.int32, (E, N), 0)
    v1 = jnp.max(probsT, axis=0, keepdims=True)
    i1 = jnp.min(jnp.where(probsT == v1, subl, E), axis=0, keepdims=True)
    probs2 = jnp.where(subl == i1, -jnp.inf, probsT)
    v2 = jnp.max(probs2, axis=0, keepdims=True)
    i2 = jnp.min(jnp.where(probs2 == v2, subl, E), axis=0, keepdims=True)

    oh0 = (subl == i1).astype(jnp.int32)
    oh1 = (subl == i2).astype(jnp.int32)
    inc0 = _lane_cumsum(oh0)
    inc1 = _lane_cumsum(oh1)
    exc0 = inc0 - oh0
    exc1 = inc1 - oh1
    tot0 = inc0[:, N - 1:N]
    tot1 = inc1[:, N - 1:N]
    counts = tot0 + tot1
    nb = (counts + (B - 1)) // B
    r8 = jax.lax.broadcasted_iota(jnp.int32, (E, E), 0)
    c8 = jax.lax.broadcasted_iota(jnp.int32, (E, E), 1)
    L8 = (r8 > c8).astype(jnp.float32)
    bs = jax.lax.dot_general(L8, nb.astype(jnp.float32),
                             (((1,), (0,)), ((), ())),
                             preferred_element_type=jnp.float32).astype(
                                 jnp.int32)
    pstart = B * bs
    slot0 = jnp.sum(oh0 * (pstart + exc0), axis=0, keepdims=True)
    slot1 = jnp.sum(oh1 * (pstart + tot0 + exc1), axis=0, keepdims=True)
    slots_ref[...] = jnp.concatenate([slot0, slot1], axis=0)
    gates_ref[...] = jnp.concatenate([v1, v2], axis=0)

    nb_total = bs[E - 1:E, 0:1] + nb[E - 1:E, 0:1]
    biota = jax.lax.broadcasted_iota(jnp.int32, (E, 32), 1)
    be = jnp.sum((biota >= bs).astype(jnp.int32), axis=0, keepdims=True) - 1
    lane32 = jax.lax.broadcasted_iota(jnp.int32, (1, 32), 1)
    meta_ref[...] = jnp.where(lane32 == NBMAX, nb_total, be)


def _routing(x, gW1, gb1, gW2, gb2):
    return pl.pallas_call(
        _routing_kernel,
        in_specs=[
            pl.BlockSpec((N, D), lambda: (0, 0)),
            pl.BlockSpec((D, E), lambda: (0, 0)),
            pl.BlockSpec((E,), lambda: (0,)),
            pl.BlockSpec((E, E), lambda: (0, 0)),
            pl.BlockSpec((E,), lambda: (0,)),
        ],
        out_specs=[
            pl.BlockSpec((K, N), lambda: (0, 0)),
            pl.BlockSpec((K, N), lambda: (0, 0)),
            pl.BlockSpec((1, 32), lambda: (0, 0)),
        ],
        out_shape=[
            jax.ShapeDtypeStruct((K, N), jnp.int32),
            jax.ShapeDtypeStruct((K, N), jnp.float32),
            jax.ShapeDtypeStruct((1, 32), jnp.int32),
        ],
    )(x, gW1, gb1, gW2, gb2)


@functools.partial(
    pl.kernel, mesh=_mesh, compiler_params=_cp,
    out_type=[jax.ShapeDtypeStruct((P, 128), jnp.float32),
              jax.ShapeDtypeStruct((P, D), jnp.float32)],
    scratch_types=[
        pltpu.VMEM((4, 32), jnp.int32),
        pltpu.VMEM((PAIRS_W,), jnp.float32),
        pltpu.VMEM((PAIRS_W, 128), jnp.float32),
        pltpu.VMEM((32, D), jnp.float32),
        pltpu.VMEM((32, D), jnp.float32),
        pltpu.SemaphoreType.DMA,
        pltpu.SemaphoreType.DMA,
    ],
)
def _dispatch_kernel(x_hbm, slots_hbm, gates_hbm, grows_hbm, xs_hbm,
                     idx_v, gv, rows_v, xb0, xb1, sem, sem2):
    wid = lax.axis_index("s") * NC + lax.axis_index("c")
    k = wid // (N // PAIRS_W)
    n0 = (wid * PAIRS_W) % N

    for j in range(4):
        pltpu.sync_copy(slots_hbm.at[k, pl.ds(n0 + j * 32, 32)], idx_v.at[j])
    pltpu.sync_copy(gates_hbm.at[k, pl.ds(n0, PAIRS_W)], gv)

    col0 = jnp.zeros((16,), jnp.int32)
    for c in range(PAIRS_W // 16):
        ridx = lax.iota(jnp.int32, 16) + c * 16
        plsc.store_scatter(rows_v, [ridx, col0], gv[pl.ds(c * 16, 16)])
    cg = []
    for j in range(4):
        cg.append(pltpu.async_copy(
            rows_v.at[pl.ds(j * 32, 32)], grows_hbm.at[idx_v.at[j]], sem2))

    bufs = (xb0, xb1)
    cps = [None, None]
    for j in range(4):
        buf = bufs[j % 2]
        if cps[j % 2] is not None:
            cps[j % 2].wait()
        pltpu.sync_copy(x_hbm.at[pl.ds(n0 + j * 32, 32)], buf)
        cps[j % 2] = pltpu.async_copy(buf, xs_hbm.at[idx_v.at[j]], sem)
    for cp in cps:
        cp.wait()
    for cp in cg:
        cp.wait()


def _gemm_kernel(meta_ref, xs_ref, gate_ref, eW1_ref, eb1_ref, eW2_ref,
                 eb2_ref, ys_ref):
    b = pl.program_id(0)
    e = meta_ref[0, b]

    @pl.when(b < meta_ref[0, NBMAX])
    def _():
        subl = jax.lax.broadcasted_iota(jnp.int32, (E, H), 0)
        b1 = jnp.sum(jnp.where(subl == e, eb1_ref[...], 0.0), axis=0,
                     keepdims=True)
        b2 = jnp.sum(jnp.where(subl == e, eb2_ref[...], 0.0), axis=0,
                     keepdims=True)
        h = jnp.tanh(
            jax.lax.dot_general(xs_ref[...].astype(jnp.bfloat16), eW1_ref[0],
                                (((1,), (0,)), ((), ())),
                                preferred_element_type=jnp.float32)
            + b1)
        y = jax.lax.dot_general(h.astype(jnp.bfloat16), eW2_ref[0],
                                (((1,), (0,)), ((), ())),
                                preferred_element_type=jnp.float32) \
            + b2
        ys_ref[...] = y * gate_ref[...][:, 0:1]


def _gemm(meta, xs, gate_rows, eW1, eb1, eW2, eb2):
    grid_spec = pltpu.PrefetchScalarGridSpec(
        num_scalar_prefetch=1,
        grid=(NBMAX,),
        in_specs=[
            pl.BlockSpec((B, D), lambda b, m: (b, 0)),
            pl.BlockSpec((B, 128), lambda b, m: (b, 0)),
            pl.BlockSpec((1, D, H), lambda b, m: (m[0, b], 0, 0)),
            pl.BlockSpec((E, H), lambda b, m: (0, 0)),
            pl.BlockSpec((1, H, D), lambda b, m: (m[0, b], 0, 0)),
            pl.BlockSpec((E, D), lambda b, m: (0, 0)),
        ],
        out_specs=pl.BlockSpec((B, D), lambda b, m: (b, 0)),
    )
    return pl.pallas_call(
        _gemm_kernel,
        grid_spec=grid_spec,
        out_shape=jax.ShapeDtypeStruct((P, D), jnp.float32),
        compiler_params=pltpu.CompilerParams(
            dimension_semantics=("arbitrary",),
        ),
    )(meta, xs, gate_rows, eW1.astype(jnp.bfloat16), eb1,
      eW2.astype(jnp.bfloat16), eb2)


@functools.partial(
    pl.kernel, mesh=_mesh, compiler_params=_cp,
    out_type=jax.ShapeDtypeStruct((N, D), jnp.float32),
    scratch_types=[
        pltpu.VMEM((4, 16), jnp.int32),
        pltpu.VMEM((4, 16), jnp.int32),
        pltpu.VMEM((16, D), jnp.float32),
        pltpu.VMEM((16, D), jnp.float32),
        pltpu.VMEM((16, D), jnp.float32),
        pltpu.VMEM((16, D), jnp.float32),
        pltpu.SemaphoreType.DMA,
        pltpu.SemaphoreType.DMA,
        pltpu.SemaphoreType.DMA,
    ],
)
def _combine_kernel(ys_hbm, slots_hbm, out_hbm, idx0, idx1,
                    a0, a1, b0, b1, sem, semb, semo):
    wid = lax.axis_index("s") * NC + lax.axis_index("c")
    n0 = wid * TOK_W
    for j in range(4):
        pltpu.sync_copy(slots_hbm.at[0, pl.ds(n0 + j * 16, 16)], idx0.at[j])
        pltpu.sync_copy(slots_hbm.at[1, pl.ds(n0 + j * 16, 16)], idx1.at[j])

    pairs = ((a0, a1, sem), (b0, b1, semb))
    gets = [None, None]
    outs = [None, None]

    def issue(j):
        p0, p1, s = pairs[j % 2]
        c0 = pltpu.async_copy(ys_hbm.at[idx0.at[j]], p0, s)
        c1 = pltpu.async_copy(ys_hbm.at[idx1.at[j]], p1, s)
        gets[j % 2] = (c0, c1)

    issue(0)
    for j in range(4):
        p0, p1, _ = pairs[j % 2]
        c0, c1 = gets[j % 2]
        c0.wait()
        c1.wait()
        if j + 1 < 4:
            if outs[(j + 1) % 2] is not None:
                outs[(j + 1) % 2].wait()
                outs[(j + 1) % 2] = None
            issue(j + 1)

        @pl.loop(0, 16)
        def _(r):
            for q in range(D // 16):
                sl = pl.ds(q * 16, 16)
                plsc.addupdate(p0.at[r, sl], p1[r, sl])

        outs[j % 2] = pltpu.async_copy(
            p0, out_hbm.at[pl.ds(n0 + j * 16, 16)], semo)
    for o in outs:
        if o is not None:
            o.wait()


@jax.jit
def _moe(x, gW1, gb1, gW2, gb2, eW1, eb1, eW2, eb2):
    slots, gates, meta = _routing(x, gW1, gb1, gW2, gb2)
    gate_rows, xs = _dispatch_kernel(x, slots, gates)
    ys = _gemm(meta, xs, gate_rows, eW1, eb1, eW2, eb2)
    return _combine_kernel(ys, slots)


def kernel(x, gW1, gb1, gW2, gb2, eW1, eb1, eW2, eb2, train):
    del train
    return _moe(x, gW1, gb1, gW2, gb2, eW1, eb1, eW2, eb2)


# speedup vs baseline: 1.1127x; 1.1127x over previous
import dataclasses
import functools

import jax
import jax.numpy as jnp
from jax import lax
from jax.experimental import pallas as pl
from jax.experimental.pallas import tpu as pltpu
from jax.experimental.pallas import tpu_sc as plsc

N, D, H, E, K = 2048, 1024, 1024, 8, 2
B = 256
NBMAX = N * K // B + E - 1
P = NBMAX * B
NC, NS = 2, 16
NW = NC * NS
PAIRS_W = N * K // NW
TOK_W = N // NW

_mesh = plsc.VectorSubcoreMesh(core_axis_name="c", subcore_axis_name="s")
_cp = pltpu.CompilerParams()
if "needs_layout_passes" in pltpu.CompilerParams.__dataclass_fields__:
    _cp = dataclasses.replace(_cp, needs_layout_passes=False)


def _lane_cumsum(x):
    r, c = x.shape
    lane = jax.lax.broadcasted_iota(jnp.int32, (r, c), 1)
    acc = x
    sh = 1
    while sh < c:
        rolled = pltpu.roll(acc, sh, axis=1)
        acc = acc + jnp.where(lane >= sh, rolled, 0)
        sh *= 2
    return acc


def _routing_kernel(x_ref, gW1_ref, gb1_ref, gW2_ref, gb2_ref,
                    slots_ref, gates_ref, meta_ref):
    x = x_ref[...]
    l1 = jnp.tanh(
        jax.lax.dot_general(x, gW1_ref[...], (((1,), (0,)), ((), ())),
                            preferred_element_type=jnp.float32)
        + gb1_ref[...].reshape(1, E))
    logits = jax.lax.dot_general(l1, gW2_ref[...], (((1,), (0,)), ((), ())),
                                 preferred_element_type=jnp.float32) \
        + gb2_ref[...].reshape(1, E)
    mx = jnp.max(logits, axis=-1, keepdims=True)
    exn = jnp.exp(logits - mx)
    probs = exn / jnp.sum(exn, axis=-1, keepdims=True)
    probsT = jnp.transpose(probs)

    subl = jax.lax.broadcasted_iota(jnp.int32, (E, N), 0)
    v1 = jnp.max(probsT, axis=0, keepdims=True)
    i1 = jnp.min(jnp.where(probsT == v1, subl, E), axis=0, keepdims=True)
    probs2 = jnp.where(subl == i1, -jnp.inf, probsT)
    v2 = jnp.max(probs2, axis=0, keepdims=True)
    i2 = jnp.min(jnp.where(probs2 == v2, subl, E), axis=0, keepdims=True)

    oh0 = (subl == i1).astype(jnp.int32)
    oh1 = (subl == i2).astype(jnp.int32)
    inc0 = _lane_cumsum(oh0)
    inc1 = _lane_cumsum(oh1)
    exc0 = inc0 - oh0
    exc1 = inc1 - oh1
    tot0 = inc0[:, N - 1:N]
    tot1 = inc1[:, N - 1:N]
    counts = tot0 + tot1
    nb = (counts + (B - 1)) // B
    r8 = jax.lax.broadcasted_iota(jnp.int32, (E, E), 0)
    c8 = jax.lax.broadcasted_iota(jnp.int32, (E, E), 1)
    L8 = (r8 > c8).astype(jnp.float32)
    bs = jax.lax.dot_general(L8, nb.astype(jnp.float32),
                             (((1,), (0,)), ((), ())),
                             preferred_element_type=jnp.float32).astype(
                                 jnp.int32)
    pstart = B * bs
    slot0 = jnp.sum(oh0 * (pstart + exc0), axis=0, keepdims=True)
    slot1 = jnp.sum(oh1 * (pstart + tot0 + exc1), axis=0, keepdims=True)
    slots_ref[...] = jnp.concatenate([slot0, slot1], axis=0)
    gates_ref[...] = jnp.concatenate([v1, v2], axis=0)

    nb_total = bs[E - 1:E, 0:1] + nb[E - 1:E, 0:1]
    biota = jax.lax.broadcasted_iota(jnp.int32, (E, 32), 1)
    be = jnp.sum((biota >= bs).astype(jnp.int32), axis=0, keepdims=True) - 1
    lane32 = jax.lax.broadcasted_iota(jnp.int32, (1, 32), 1)
    meta_ref[...] = jnp.where(lane32 == NBMAX, nb_total, be)


def _routing(x, gW1, gb1, gW2, gb2):
    return pl.pallas_call(
        _routing_kernel,
        in_specs=[
            pl.BlockSpec((N, D), lambda: (0, 0)),
            pl.BlockSpec((D, E), lambda: (0, 0)),
            pl.BlockSpec((E,), lambda: (0,)),
            pl.BlockSpec((E, E), lambda: (0, 0)),
            pl.BlockSpec((E,), lambda: (0,)),
        ],
        out_specs=[
            pl.BlockSpec((K, N), lambda: (0, 0)),
            pl.BlockSpec((K, N), lambda: (0, 0)),
            pl.BlockSpec((1, 32), lambda: (0, 0)),
        ],
        out_shape=[
            jax.ShapeDtypeStruct((K, N), jnp.int32),
            jax.ShapeDtypeStruct((K, N), jnp.float32),
            jax.ShapeDtypeStruct((1, 32), jnp.int32),
        ],
    )(x, gW1, gb1, gW2, gb2)


@functools.partial(
    pl.kernel, mesh=_mesh, compiler_params=_cp,
    out_type=[jax.ShapeDtypeStruct((P, 128), jnp.float32),
              jax.ShapeDtypeStruct((P, D), jnp.float32)],
    scratch_types=[
        pltpu.VMEM((4, 32), jnp.int32),
        pltpu.VMEM((PAIRS_W,), jnp.float32),
        pltpu.VMEM((PAIRS_W, 128), jnp.float32),
        pltpu.VMEM((32, D), jnp.float32),
        pltpu.VMEM((32, D), jnp.float32),
        pltpu.SemaphoreType.DMA,
        pltpu.SemaphoreType.DMA,
    ],
)
def _dispatch_kernel(x_hbm, slots_hbm, gates_hbm, grows_hbm, xs_hbm,
                     idx_v, gv, rows_v, xb0, xb1, sem, sem2):
    wid = lax.axis_index("s") * NC + lax.axis_index("c")
    k = wid // (N // PAIRS_W)
    n0 = (wid * PAIRS_W) % N

    for j in range(4):
        pltpu.sync_copy(slots_hbm.at[k, pl.ds(n0 + j * 32, 32)], idx_v.at[j])
    pltpu.sync_copy(gates_hbm.at[k, pl.ds(n0, PAIRS_W)], gv)

    col0 = jnp.zeros((16,), jnp.int32)
    for c in range(PAIRS_W // 16):
        ridx = lax.iota(jnp.int32, 16) + c * 16
        plsc.store_scatter(rows_v, [ridx, col0], gv[pl.ds(c * 16, 16)])
    cg = []
    for j in range(4):
        cg.append(pltpu.async_copy(
            rows_v.at[pl.ds(j * 32, 32)], grows_hbm.at[idx_v.at[j]], sem2))

    bufs = (xb0, xb1)
    cps = [None, None]
    for j in range(4):
        buf = bufs[j % 2]
        if cps[j % 2] is not None:
            cps[j % 2].wait()
        pltpu.sync_copy(x_hbm.at[pl.ds(n0 + j * 32, 32)], buf)
        cps[j % 2] = pltpu.async_copy(buf, xs_hbm.at[idx_v.at[j]], sem)
    for cp in cps:
        cp.wait()
    for cp in cg:
        cp.wait()


def _gemm_kernel(meta_ref, xs_ref, gate_ref, eW1_ref, eb1_ref, eW2_ref,
                 eb2_ref, ys_ref, w1bf, w2bf):
    b = pl.program_id(0)
    e = meta_ref[0, b]
    prev = meta_ref[0, jnp.maximum(b - 1, 0)]

    @pl.when((b == 0) | (e != prev))
    def _cast():
        w1bf[...] = eW1_ref[0].astype(jnp.bfloat16)
        w2bf[...] = eW2_ref[0].astype(jnp.bfloat16)

    @pl.when(b < meta_ref[0, NBMAX])
    def _():
        subl = jax.lax.broadcasted_iota(jnp.int32, (E, H), 0)
        b1 = jnp.sum(jnp.where(subl == e, eb1_ref[...], 0.0), axis=0,
                     keepdims=True)
        b2 = jnp.sum(jnp.where(subl == e, eb2_ref[...], 0.0), axis=0,
                     keepdims=True)
        h = jnp.tanh(
            jax.lax.dot_general(xs_ref[...].astype(jnp.bfloat16), w1bf[...],
                                (((1,), (0,)), ((), ())),
                                preferred_element_type=jnp.float32)
            + b1)
        y = jax.lax.dot_general(h.astype(jnp.bfloat16), w2bf[...],
                                (((1,), (0,)), ((), ())),
                                preferred_element_type=jnp.float32) \
            + b2
        ys_ref[...] = y * gate_ref[...][:, 0:1]


def _gemm(meta, xs, gate_rows, eW1, eb1, eW2, eb2):
    grid_spec = pltpu.PrefetchScalarGridSpec(
        num_scalar_prefetch=1,
        grid=(NBMAX,),
        in_specs=[
            pl.BlockSpec((B, D), lambda b, m: (b, 0)),
            pl.BlockSpec((B, 128), lambda b, m: (b, 0)),
            pl.BlockSpec((1, D, H), lambda b, m: (m[0, b], 0, 0)),
            pl.BlockSpec((E, H), lambda b, m: (0, 0)),
            pl.BlockSpec((1, H, D), lambda b, m: (m[0, b], 0, 0)),
            pl.BlockSpec((E, D), lambda b, m: (0, 0)),
        ],
        out_specs=pl.BlockSpec((B, D), lambda b, m: (b, 0)),
        scratch_shapes=[
            pltpu.VMEM((D, H), jnp.bfloat16),
            pltpu.VMEM((H, D), jnp.bfloat16),
        ],
    )
    return pl.pallas_call(
        _gemm_kernel,
        grid_spec=grid_spec,
        out_shape=jax.ShapeDtypeStruct((P, D), jnp.float32),
        compiler_params=pltpu.CompilerParams(
            dimension_semantics=("arbitrary",),
        ),
    )(meta, xs, gate_rows, eW1, eb1, eW2, eb2)


@functools.partial(
    pl.kernel, mesh=_mesh, compiler_params=_cp,
    out_type=jax.ShapeDtypeStruct((N, D), jnp.float32),
    scratch_types=[
        pltpu.VMEM((4, 16), jnp.int32),
        pltpu.VMEM((4, 16), jnp.int32),
        pltpu.VMEM((16, D), jnp.float32),
        pltpu.VMEM((16, D), jnp.float32),
        pltpu.VMEM((16, D), jnp.float32),
        pltpu.VMEM((16, D), jnp.float32),
        pltpu.SemaphoreType.DMA,
        pltpu.SemaphoreType.DMA,
        pltpu.SemaphoreType.DMA,
    ],
)
def _combine_kernel(ys_hbm, slots_hbm, out_hbm, idx0, idx1,
                    a0, a1, b0, b1, sem, semb, semo):
    wid = lax.axis_index("s") * NC + lax.axis_index("c")
    n0 = wid * TOK_W
    for j in range(4):
        pltpu.sync_copy(slots_hbm.at[0, pl.ds(n0 + j * 16, 16)], idx0.at[j])
        pltpu.sync_copy(slots_hbm.at[1, pl.ds(n0 + j * 16, 16)], idx1.at[j])

    pairs = ((a0, a1, sem), (b0, b1, semb))
    gets = [None, None]
    outs = [None, None]

    def issue(j):
        p0, p1, s = pairs[j % 2]
        c0 = pltpu.async_copy(ys_hbm.at[idx0.at[j]], p0, s)
        c1 = pltpu.async_copy(ys_hbm.at[idx1.at[j]], p1, s)
        gets[j % 2] = (c0, c1)

    issue(0)
    for j in range(4):
        p0, p1, _ = pairs[j % 2]
        c0, c1 = gets[j % 2]
        c0.wait()
        c1.wait()
        if j + 1 < 4:
            if outs[(j + 1) % 2] is not None:
                outs[(j + 1) % 2].wait()
                outs[(j + 1) % 2] = None
            issue(j + 1)

        @pl.loop(0, 16)
        def _(r):
            for q in range(D // 16):
                sl = pl.ds(q * 16, 16)
                plsc.addupdate(p0.at[r, sl], p1[r, sl])

        outs[j % 2] = pltpu.async_copy(
            p0, out_hbm.at[pl.ds(n0 + j * 16, 16)], semo)
    for o in outs:
        if o is not None:
            o.wait()


@jax.jit
def _moe(x, gW1, gb1, gW2, gb2, eW1, eb1, eW2, eb2):
    slots, gates, meta = _routing(x, gW1, gb1, gW2, gb2)
    gate_rows, xs = _dispatch_kernel(x, slots, gates)
    ys = _gemm(meta, xs, gate_rows, eW1, eb1, eW2, eb2)
    return _combine_kernel(ys, slots)


def kernel(x, gW1, gb1, gW2, gb2, eW1, eb1, eW2, eb2, train):
    del train
    return _moe(x, gW1, gb1, gW2, gb2, eW1, eb1, eW2, eb2)


# speedup vs baseline: 1.1522x; 1.0355x over previous
import dataclasses
import functools

import jax
import jax.numpy as jnp
from jax import lax
from jax.experimental import pallas as pl
from jax.experimental.pallas import tpu as pltpu
from jax.experimental.pallas import tpu_sc as plsc

N, D, H, E, K = 2048, 1024, 1024, 8, 2
B = 512
NBMAX = N * K // B + E - 1
P = NBMAX * B
NC, NS = 2, 16
NW = NC * NS
PAIRS_W = N * K // NW
TOK_W = N // NW

_mesh = plsc.VectorSubcoreMesh(core_axis_name="c", subcore_axis_name="s")
_cp = pltpu.CompilerParams()
if "needs_layout_passes" in pltpu.CompilerParams.__dataclass_fields__:
    _cp = dataclasses.replace(_cp, needs_layout_passes=False)


def _lane_cumsum(x):
    r, c = x.shape
    lane = jax.lax.broadcasted_iota(jnp.int32, (r, c), 1)
    acc = x
    sh = 1
    while sh < c:
        rolled = pltpu.roll(acc, sh, axis=1)
        acc = acc + jnp.where(lane >= sh, rolled, 0)
        sh *= 2
    return acc


def _routing_kernel(x_ref, gW1_ref, gb1_ref, gW2_ref, gb2_ref,
                    slots_ref, gates_ref, meta_ref):
    x = x_ref[...]
    l1 = jnp.tanh(
        jax.lax.dot_general(x, gW1_ref[...], (((1,), (0,)), ((), ())),
                            preferred_element_type=jnp.float32)
        + gb1_ref[...].reshape(1, E))
    logits = jax.lax.dot_general(l1, gW2_ref[...], (((1,), (0,)), ((), ())),
                                 preferred_element_type=jnp.float32) \
        + gb2_ref[...].reshape(1, E)
    mx = jnp.max(logits, axis=-1, keepdims=True)
    exn = jnp.exp(logits - mx)
    probs = exn / jnp.sum(exn, axis=-1, keepdims=True)
    probsT = jnp.transpose(probs)

    subl = jax.lax.broadcasted_iota(jnp.int32, (E, N), 0)
    v1 = jnp.max(probsT, axis=0, keepdims=True)
    i1 = jnp.min(jnp.where(probsT == v1, subl, E), axis=0, keepdims=True)
    probs2 = jnp.where(subl == i1, -jnp.inf, probsT)
    v2 = jnp.max(probs2, axis=0, keepdims=True)
    i2 = jnp.min(jnp.where(probs2 == v2, subl, E), axis=0, keepdims=True)

    oh0 = (subl == i1).astype(jnp.int32)
    oh1 = (subl == i2).astype(jnp.int32)
    inc0 = _lane_cumsum(oh0)
    inc1 = _lane_cumsum(oh1)
    exc0 = inc0 - oh0
    exc1 = inc1 - oh1
    tot0 = inc0[:, N - 1:N]
    tot1 = inc1[:, N - 1:N]
    counts = tot0 + tot1
    nb = (counts + (B - 1)) // B
    r8 = jax.lax.broadcasted_iota(jnp.int32, (E, E), 0)
    c8 = jax.lax.broadcasted_iota(jnp.int32, (E, E), 1)
    L8 = (r8 > c8).astype(jnp.float32)
    bs = jax.lax.dot_general(L8, nb.astype(jnp.float32),
                             (((1,), (0,)), ((), ())),
                             preferred_element_type=jnp.float32).astype(
                                 jnp.int32)
    pstart = B * bs
    slot0 = jnp.sum(oh0 * (pstart + exc0), axis=0, keepdims=True)
    slot1 = jnp.sum(oh1 * (pstart + tot0 + exc1), axis=0, keepdims=True)
    slots_ref[...] = jnp.concatenate([slot0, slot1], axis=0)
    gates_ref[...] = jnp.concatenate([v1, v2], axis=0)

    nb_total = bs[E - 1:E, 0:1] + nb[E - 1:E, 0:1]
    biota = jax.lax.broadcasted_iota(jnp.int32, (E, 32), 1)
    be = jnp.sum((biota >= bs).astype(jnp.int32), axis=0, keepdims=True) - 1
    lane32 = jax.lax.broadcasted_iota(jnp.int32, (1, 32), 1)
    meta_ref[...] = jnp.where(lane32 == NBMAX, nb_total, be)


def _routing(x, gW1, gb1, gW2, gb2):
    return pl.pallas_call(
        _routing_kernel,
        in_specs=[
            pl.BlockSpec((N, D), lambda: (0, 0)),
            pl.BlockSpec((D, E), lambda: (0, 0)),
            pl.BlockSpec((E,), lambda: (0,)),
            pl.BlockSpec((E, E), lambda: (0, 0)),
            pl.BlockSpec((E,), lambda: (0,)),
        ],
        out_specs=[
            pl.BlockSpec((K, N), lambda: (0, 0)),
            pl.BlockSpec((K, N), lambda: (0, 0)),
            pl.BlockSpec((1, 32), lambda: (0, 0)),
        ],
        out_shape=[
            jax.ShapeDtypeStruct((K, N), jnp.int32),
            jax.ShapeDtypeStruct((K, N), jnp.float32),
            jax.ShapeDtypeStruct((1, 32), jnp.int32),
        ],
    )(x, gW1, gb1, gW2, gb2)


@functools.partial(
    pl.kernel, mesh=_mesh, compiler_params=_cp,
    out_type=[jax.ShapeDtypeStruct((P, 128), jnp.float32),
              jax.ShapeDtypeStruct((P, D), jnp.float32)],
    scratch_types=[
        pltpu.VMEM((4, 32), jnp.int32),
        pltpu.VMEM((PAIRS_W,), jnp.float32),
        pltpu.VMEM((PAIRS_W, 128), jnp.float32),
        pltpu.VMEM((32, D), jnp.float32),
        pltpu.VMEM((32, D), jnp.float32),
        pltpu.SemaphoreType.DMA,
        pltpu.SemaphoreType.DMA,
    ],
)
def _dispatch_kernel(x_hbm, slots_hbm, gates_hbm, grows_hbm, xs_hbm,
                     idx_v, gv, rows_v, xb0, xb1, sem, sem2):
    wid = lax.axis_index("s") * NC + lax.axis_index("c")
    k = wid // (N // PAIRS_W)
    n0 = (wid * PAIRS_W) % N

    for j in range(4):
        pltpu.sync_copy(slots_hbm.at[k, pl.ds(n0 + j * 32, 32)], idx_v.at[j])
    pltpu.sync_copy(gates_hbm.at[k, pl.ds(n0, PAIRS_W)], gv)

    col0 = jnp.zeros((16,), jnp.int32)
    for c in range(PAIRS_W // 16):
        ridx = lax.iota(jnp.int32, 16) + c * 16
        plsc.store_scatter(rows_v, [ridx, col0], gv[pl.ds(c * 16, 16)])
    cg = []
    for j in range(4):
        cg.append(pltpu.async_copy(
            rows_v.at[pl.ds(j * 32, 32)], grows_hbm.at[idx_v.at[j]], sem2))

    bufs = (xb0, xb1)
    cps = [None, None]
    for j in range(4):
        buf = bufs[j % 2]
        if cps[j % 2] is not None:
            cps[j % 2].wait()
        pltpu.sync_copy(x_hbm.at[pl.ds(n0 + j * 32, 32)], buf)
        cps[j % 2] = pltpu.async_copy(buf, xs_hbm.at[idx_v.at[j]], sem)
    for cp in cps:
        cp.wait()
    for cp in cg:
        cp.wait()


def _gemm_kernel(meta_ref, xs_ref, gate_ref, eW1_ref, eb1_ref, eW2_ref,
                 eb2_ref, ys_ref, w1bf, w2bf):
    b = pl.program_id(0)
    e = meta_ref[0, b]
    prev = meta_ref[0, jnp.maximum(b - 1, 0)]

    @pl.when((b == 0) | (e != prev))
    def _cast():
        w1bf[...] = eW1_ref[0].astype(jnp.bfloat16)
        w2bf[...] = eW2_ref[0].astype(jnp.bfloat16)

    @pl.when(b < meta_ref[0, NBMAX])
    def _():
        subl = jax.lax.broadcasted_iota(jnp.int32, (E, H), 0)
        b1 = jnp.sum(jnp.where(subl == e, eb1_ref[...], 0.0), axis=0,
                     keepdims=True)
        b2 = jnp.sum(jnp.where(subl == e, eb2_ref[...], 0.0), axis=0,
                     keepdims=True)
        h = jnp.tanh(
            jax.lax.dot_general(xs_ref[...].astype(jnp.bfloat16), w1bf[...],
                                (((1,), (0,)), ((), ())),
                                preferred_element_type=jnp.float32)
            + b1)
        y = jax.lax.dot_general(h.astype(jnp.bfloat16), w2bf[...],
                                (((1,), (0,)), ((), ())),
                                preferred_element_type=jnp.float32) \
            + b2
        ys_ref[...] = y * gate_ref[...][:, 0:1]


def _gemm(meta, xs, gate_rows, eW1, eb1, eW2, eb2):
    grid_spec = pltpu.PrefetchScalarGridSpec(
        num_scalar_prefetch=1,
        grid=(NBMAX,),
        in_specs=[
            pl.BlockSpec((B, D), lambda b, m: (b, 0)),
            pl.BlockSpec((B, 128), lambda b, m: (b, 0)),
            pl.BlockSpec((1, D, H), lambda b, m: (m[0, b], 0, 0)),
            pl.BlockSpec((E, H), lambda b, m: (0, 0)),
            pl.BlockSpec((1, H, D), lambda b, m: (m[0, b], 0, 0)),
            pl.BlockSpec((E, D), lambda b, m: (0, 0)),
        ],
        out_specs=pl.BlockSpec((B, D), lambda b, m: (b, 0)),
        scratch_shapes=[
            pltpu.VMEM((D, H), jnp.bfloat16),
            pltpu.VMEM((H, D), jnp.bfloat16),
        ],
    )
    return pl.pallas_call(
        _gemm_kernel,
        grid_spec=grid_spec,
        out_shape=jax.ShapeDtypeStruct((P, D), jnp.float32),
        compiler_params=pltpu.CompilerParams(
            dimension_semantics=("arbitrary",),
        ),
    )(meta, xs, gate_rows, eW1, eb1, eW2, eb2)


@functools.partial(
    pl.kernel, mesh=_mesh, compiler_params=_cp,
    out_type=jax.ShapeDtypeStruct((N, D), jnp.float32),
    scratch_types=[
        pltpu.VMEM((4, 16), jnp.int32),
        pltpu.VMEM((4, 16), jnp.int32),
        pltpu.VMEM((16, D), jnp.float32),
        pltpu.VMEM((16, D), jnp.float32),
        pltpu.VMEM((16, D), jnp.float32),
        pltpu.VMEM((16, D), jnp.float32),
        pltpu.SemaphoreType.DMA,
        pltpu.SemaphoreType.DMA,
        pltpu.SemaphoreType.DMA,
    ],
)
def _combine_kernel(ys_hbm, slots_hbm, out_hbm, idx0, idx1,
                    a0, a1, b0, b1, sem, semb, semo):
    wid = lax.axis_index("s") * NC + lax.axis_index("c")
    n0 = wid * TOK_W
    for j in range(4):
        pltpu.sync_copy(slots_hbm.at[0, pl.ds(n0 + j * 16, 16)], idx0.at[j])
        pltpu.sync_copy(slots_hbm.at[1, pl.ds(n0 + j * 16, 16)], idx1.at[j])

    pairs = ((a0, a1, sem), (b0, b1, semb))
    gets = [None, None]
    outs = [None, None]

    def issue(j):
        p0, p1, s = pairs[j % 2]
        c0 = pltpu.async_copy(ys_hbm.at[idx0.at[j]], p0, s)
        c1 = pltpu.async_copy(ys_hbm.at[idx1.at[j]], p1, s)
        gets[j % 2] = (c0, c1)

    issue(0)
    for j in range(4):
        p0, p1, _ = pairs[j % 2]
        c0, c1 = gets[j % 2]
        c0.wait()
        c1.wait()
        if j + 1 < 4:
            if outs[(j + 1) % 2] is not None:
                outs[(j + 1) % 2].wait()
                outs[(j + 1) % 2] = None
            issue(j + 1)

        @pl.loop(0, 16)
        def _(r):
            for q in range(D // 16):
                sl = pl.ds(q * 16, 16)
                plsc.addupdate(p0.at[r, sl], p1[r, sl])

        outs[j % 2] = pltpu.async_copy(
            p0, out_hbm.at[pl.ds(n0 + j * 16, 16)], semo)
    for o in outs:
        if o is not None:
            o.wait()


@jax.jit
def _moe(x, gW1, gb1, gW2, gb2, eW1, eb1, eW2, eb2):
    slots, gates, meta = _routing(x, gW1, gb1, gW2, gb2)
    gate_rows, xs = _dispatch_kernel(x, slots, gates)
    ys = _gemm(meta, xs, gate_rows, eW1, eb1, eW2, eb2)
    return _combine_kernel(ys, slots)


def kernel(x, gW1, gb1, gW2, gb2, eW1, eb1, eW2, eb2, train):
    del train
    return _moe(x, gW1, gb1, gW2, gb2, eW1, eb1, eW2, eb2)


# speedup vs baseline: 1.1748x; 1.0196x over previous
import dataclasses
import functools

import jax
import jax.numpy as jnp
from jax import lax
from jax.experimental import pallas as pl
from jax.experimental.pallas import tpu as pltpu
from jax.experimental.pallas import tpu_sc as plsc

N, D, H, E, K = 2048, 1024, 1024, 8, 2
B = 512
NBMAX = N * K // B + E - 1
P = NBMAX * B
NC, NS = 2, 16
NW = NC * NS
PAIRS_W = N * K // NW
TOK_W = N // NW

_mesh = plsc.VectorSubcoreMesh(core_axis_name="c", subcore_axis_name="s")
_cp = pltpu.CompilerParams()
if "needs_layout_passes" in pltpu.CompilerParams.__dataclass_fields__:
    _cp = dataclasses.replace(_cp, needs_layout_passes=False)


def _lane_cumsum(x):
    r, c = x.shape
    lane = jax.lax.broadcasted_iota(jnp.int32, (r, c), 1)
    acc = x
    sh = 1
    while sh < c:
        rolled = pltpu.roll(acc, sh, axis=1)
        acc = acc + jnp.where(lane >= sh, rolled, 0)
        sh *= 2
    return acc


def _routing_kernel(x_ref, gW1_ref, gb1_ref, gW2_ref, gb2_ref,
                    slots_ref, gates_ref, meta_ref):
    x = x_ref[...]
    l1 = jnp.tanh(
        jax.lax.dot_general(x, gW1_ref[...], (((1,), (0,)), ((), ())),
                            preferred_element_type=jnp.float32)
        + gb1_ref[...].reshape(1, E))
    logits = jax.lax.dot_general(l1, gW2_ref[...], (((1,), (0,)), ((), ())),
                                 preferred_element_type=jnp.float32) \
        + gb2_ref[...].reshape(1, E)
    mx = jnp.max(logits, axis=-1, keepdims=True)
    exn = jnp.exp(logits - mx)
    probs = exn / jnp.sum(exn, axis=-1, keepdims=True)
    probsT = jnp.transpose(probs)

    subl = jax.lax.broadcasted_iota(jnp.int32, (E, N), 0)
    v1 = jnp.max(probsT, axis=0, keepdims=True)
    i1 = jnp.min(jnp.where(probsT == v1, subl, E), axis=0, keepdims=True)
    probs2 = jnp.where(subl == i1, -jnp.inf, probsT)
    v2 = jnp.max(probs2, axis=0, keepdims=True)
    i2 = jnp.min(jnp.where(probs2 == v2, subl, E), axis=0, keepdims=True)

    oh0 = (subl == i1).astype(jnp.int32)
    oh1 = (subl == i2).astype(jnp.int32)
    inc0 = _lane_cumsum(oh0)
    inc1 = _lane_cumsum(oh1)
    exc0 = inc0 - oh0
    exc1 = inc1 - oh1
    tot0 = inc0[:, N - 1:N]
    tot1 = inc1[:, N - 1:N]
    counts = tot0 + tot1
    nb = (counts + (B - 1)) // B
    r8 = jax.lax.broadcasted_iota(jnp.int32, (E, E), 0)
    c8 = jax.lax.broadcasted_iota(jnp.int32, (E, E), 1)
    L8 = (r8 > c8).astype(jnp.float32)
    bs = jax.lax.dot_general(L8, nb.astype(jnp.float32),
                             (((1,), (0,)), ((), ())),
                             preferred_element_type=jnp.float32).astype(
                                 jnp.int32)
    pstart = B * bs
    slot0 = jnp.sum(oh0 * (pstart + exc0), axis=0, keepdims=True)
    slot1 = jnp.sum(oh1 * (pstart + tot0 + exc1), axis=0, keepdims=True)
    slots_ref[...] = jnp.concatenate([slot0, slot1], axis=0)
    gates_ref[...] = jnp.concatenate([v1, v2], axis=0)

    nb_total = bs[E - 1:E, 0:1] + nb[E - 1:E, 0:1]
    biota = jax.lax.broadcasted_iota(jnp.int32, (E, 32), 1)
    be = jnp.sum((biota >= bs).astype(jnp.int32), axis=0, keepdims=True) - 1
    lane32 = jax.lax.broadcasted_iota(jnp.int32, (1, 32), 1)
    meta_ref[...] = jnp.where(lane32 == NBMAX, nb_total, be)


def _routing(x, gW1, gb1, gW2, gb2):
    return pl.pallas_call(
        _routing_kernel,
        in_specs=[
            pl.BlockSpec((N, D), lambda: (0, 0)),
            pl.BlockSpec((D, E), lambda: (0, 0)),
            pl.BlockSpec((E,), lambda: (0,)),
            pl.BlockSpec((E, E), lambda: (0, 0)),
            pl.BlockSpec((E,), lambda: (0,)),
        ],
        out_specs=[
            pl.BlockSpec((K, N), lambda: (0, 0)),
            pl.BlockSpec((K, N), lambda: (0, 0)),
            pl.BlockSpec((1, 32), lambda: (0, 0)),
        ],
        out_shape=[
            jax.ShapeDtypeStruct((K, N), jnp.int32),
            jax.ShapeDtypeStruct((K, N), jnp.float32),
            jax.ShapeDtypeStruct((1, 32), jnp.int32),
        ],
    )(x, gW1, gb1, gW2, gb2)


@functools.partial(
    pl.kernel, mesh=_mesh, compiler_params=_cp,
    out_type=[jax.ShapeDtypeStruct((P, 128), jnp.float32),
              jax.ShapeDtypeStruct((P, D), jnp.float32)],
    scratch_types=[
        pltpu.VMEM((4, 32), jnp.int32),
        pltpu.VMEM((PAIRS_W,), jnp.float32),
        pltpu.VMEM((PAIRS_W, 128), jnp.float32),
        pltpu.VMEM((32, D), jnp.float32),
        pltpu.VMEM((32, D), jnp.float32),
        pltpu.SemaphoreType.DMA,
        pltpu.SemaphoreType.DMA,
    ],
)
def _dispatch_kernel(x_hbm, slots_hbm, gates_hbm, grows_hbm, xs_hbm,
                     idx_v, gv, rows_v, xb0, xb1, sem, sem2):
    wid = lax.axis_index("s") * NC + lax.axis_index("c")
    k = wid // (N // PAIRS_W)
    n0 = (wid * PAIRS_W) % N

    for j in range(4):
        pltpu.sync_copy(slots_hbm.at[k, pl.ds(n0 + j * 32, 32)], idx_v.at[j])
    pltpu.sync_copy(gates_hbm.at[k, pl.ds(n0, PAIRS_W)], gv)

    col0 = jnp.zeros((16,), jnp.int32)
    for c in range(PAIRS_W // 16):
        ridx = lax.iota(jnp.int32, 16) + c * 16
        plsc.store_scatter(rows_v, [ridx, col0], gv[pl.ds(c * 16, 16)])
    cg = []
    for j in range(4):
        cg.append(pltpu.async_copy(
            rows_v.at[pl.ds(j * 32, 32)], grows_hbm.at[idx_v.at[j]], sem2))

    bufs = (xb0, xb1)
    cps = [None, None]
    for j in range(4):
        buf = bufs[j % 2]
        if cps[j % 2] is not None:
            cps[j % 2].wait()
        pltpu.sync_copy(x_hbm.at[pl.ds(n0 + j * 32, 32)], buf)
        cps[j % 2] = pltpu.async_copy(buf, xs_hbm.at[idx_v.at[j]], sem)
    for cp in cps:
        cp.wait()
    for cp in cg:
        cp.wait()


CHUNKS = 4
CW = H // CHUNKS


def _gemm_kernel(meta_ref, xs_ref, gate_ref, eW1_ref, eb1_ref, eW2_ref,
                 eb2_ref, ys_ref):
    b = pl.program_id(0)
    e = meta_ref[0, b]

    @pl.when(b < meta_ref[0, NBMAX])
    def _():
        subl = jax.lax.broadcasted_iota(jnp.int32, (E, H), 0)
        b1 = jnp.sum(jnp.where(subl == e, eb1_ref[...], 0.0), axis=0,
                     keepdims=True)
        b2 = jnp.sum(jnp.where(subl == e, eb2_ref[...], 0.0), axis=0,
                     keepdims=True)
        xb = xs_ref[...].astype(jnp.bfloat16)
        hs = []
        for c in range(CHUNKS):
            w1c = eW1_ref[0][:, c * CW:(c + 1) * CW].astype(jnp.bfloat16)
            hc = jax.lax.dot_general(xb, w1c, (((1,), (0,)), ((), ())),
                                     preferred_element_type=jnp.float32)
            hs.append(jnp.tanh(hc + b1[:, c * CW:(c + 1) * CW])
                      .astype(jnp.bfloat16))
        hb = jnp.concatenate(hs, axis=1)
        g = gate_ref[...][:, 0:1]
        for c in range(CHUNKS):
            w2c = eW2_ref[0][:, c * CW:(c + 1) * CW].astype(jnp.bfloat16)
            yc = jax.lax.dot_general(hb, w2c, (((1,), (0,)), ((), ())),
                                     preferred_element_type=jnp.float32)
            ys_ref[:, c * CW:(c + 1) * CW] = \
                (yc + b2[:, c * CW:(c + 1) * CW]) * g


def _gemm(meta, xs, gate_rows, eW1, eb1, eW2, eb2):
    grid_spec = pltpu.PrefetchScalarGridSpec(
        num_scalar_prefetch=1,
        grid=(NBMAX,),
        in_specs=[
            pl.BlockSpec((B, D), lambda b, m: (b, 0)),
            pl.BlockSpec((B, 128), lambda b, m: (b, 0)),
            pl.BlockSpec((1, D, H), lambda b, m: (m[0, b], 0, 0)),
            pl.BlockSpec((E, H), lambda b, m: (0, 0)),
            pl.BlockSpec((1, H, D), lambda b, m: (m[0, b], 0, 0)),
            pl.BlockSpec((E, D), lambda b, m: (0, 0)),
        ],
        out_specs=pl.BlockSpec((B, D), lambda b, m: (b, 0)),
    )
    return pl.pallas_call(
        _gemm_kernel,
        grid_spec=grid_spec,
        out_shape=jax.ShapeDtypeStruct((P, D), jnp.float32),
        compiler_params=pltpu.CompilerParams(
            dimension_semantics=("arbitrary",),
        ),
    )(meta, xs, gate_rows, eW1, eb1, eW2, eb2)


@functools.partial(
    pl.kernel, mesh=_mesh, compiler_params=_cp,
    out_type=jax.ShapeDtypeStruct((N, D), jnp.float32),
    scratch_types=[
        pltpu.VMEM((4, 16), jnp.int32),
        pltpu.VMEM((4, 16), jnp.int32),
        pltpu.VMEM((16, D), jnp.float32),
        pltpu.VMEM((16, D), jnp.float32),
        pltpu.VMEM((16, D), jnp.float32),
        pltpu.VMEM((16, D), jnp.float32),
        pltpu.SemaphoreType.DMA,
        pltpu.SemaphoreType.DMA,
        pltpu.SemaphoreType.DMA,
    ],
)
def _combine_kernel(ys_hbm, slots_hbm, out_hbm, idx0, idx1,
                    a0, a1, b0, b1, sem, semb, semo):
    wid = lax.axis_index("s") * NC + lax.axis_index("c")
    n0 = wid * TOK_W
    for j in range(4):
        pltpu.sync_copy(slots_hbm.at[0, pl.ds(n0 + j * 16, 16)], idx0.at[j])
        pltpu.sync_copy(slots_hbm.at[1, pl.ds(n0 + j * 16, 16)], idx1.at[j])

    pairs = ((a0, a1, sem), (b0, b1, semb))
    gets = [None, None]
    outs = [None, None]

    def issue(j):
        p0, p1, s = pairs[j % 2]
        c0 = pltpu.async_copy(ys_hbm.at[idx0.at[j]], p0, s)
        c1 = pltpu.async_copy(ys_hbm.at[idx1.at[j]], p1, s)
        gets[j % 2] = (c0, c1)

    issue(0)
    for j in range(4):
        p0, p1, _ = pairs[j % 2]
        c0, c1 = gets[j % 2]
        c0.wait()
        c1.wait()
        if j + 1 < 4:
            if outs[(j + 1) % 2] is not None:
                outs[(j + 1) % 2].wait()
                outs[(j + 1) % 2] = None
            issue(j + 1)

        @pl.loop(0, 16)
        def _(r):
            for q in range(D // 16):
                sl = pl.ds(q * 16, 16)
                plsc.addupdate(p0.at[r, sl], p1[r, sl])

        outs[j % 2] = pltpu.async_copy(
            p0, out_hbm.at[pl.ds(n0 + j * 16, 16)], semo)
    for o in outs:
        if o is not None:
            o.wait()


@jax.jit
def _moe(x, gW1, gb1, gW2, gb2, eW1, eb1, eW2, eb2):
    slots, gates, meta = _routing(x, gW1, gb1, gW2, gb2)
    gate_rows, xs = _dispatch_kernel(x, slots, gates)
    ys = _gemm(meta, xs, gate_rows, eW1, eb1, eW2, eb2)
    return _combine_kernel(ys, slots)


def kernel(x, gW1, gb1, gW2, gb2, eW1, eb1, eW2, eb2, train):
    del train
    return _moe(x, gW1, gb1, gW2, gb2, eW1, eb1, eW2, eb2)


# speedup vs baseline: 1.2154x; 1.0345x over previous
import dataclasses
import functools

import jax
import jax.numpy as jnp
from jax import lax
from jax.experimental import pallas as pl
from jax.experimental.pallas import tpu as pltpu
from jax.experimental.pallas import tpu_sc as plsc

N, D, H, E, K = 2048, 1024, 1024, 8, 2
B = 512
NBMAX = N * K // B + E - 1
P = NBMAX * B
NC, NS = 2, 16
NW = NC * NS
PAIRS_W = N * K // NW
TOK_W = N // NW

_mesh = plsc.VectorSubcoreMesh(core_axis_name="c", subcore_axis_name="s")
_cp = pltpu.CompilerParams()
if "needs_layout_passes" in pltpu.CompilerParams.__dataclass_fields__:
    _cp = dataclasses.replace(_cp, needs_layout_passes=False)


def _lane_cumsum(x):
    r, c = x.shape
    lane = jax.lax.broadcasted_iota(jnp.int32, (r, c), 1)
    acc = x
    sh = 1
    while sh < c:
        rolled = pltpu.roll(acc, sh, axis=1)
        acc = acc + jnp.where(lane >= sh, rolled, 0)
        sh *= 2
    return acc


def _routing_kernel(x_ref, gW1_ref, gb1_ref, gW2_ref, gb2_ref,
                    slots_ref, gates_ref, meta_ref):
    x = x_ref[...]
    l1 = jnp.tanh(
        jax.lax.dot_general(x, gW1_ref[...], (((1,), (0,)), ((), ())),
                            preferred_element_type=jnp.float32)
        + gb1_ref[...].reshape(1, E))
    logits = jax.lax.dot_general(l1, gW2_ref[...], (((1,), (0,)), ((), ())),
                                 preferred_element_type=jnp.float32) \
        + gb2_ref[...].reshape(1, E)
    mx = jnp.max(logits, axis=-1, keepdims=True)
    exn = jnp.exp(logits - mx)
    probs = exn / jnp.sum(exn, axis=-1, keepdims=True)
    probsT = jnp.transpose(probs)

    subl = jax.lax.broadcasted_iota(jnp.int32, (E, N), 0)
    v1 = jnp.max(probsT, axis=0, keepdims=True)
    i1 = jnp.min(jnp.where(probsT == v1, subl, E), axis=0, keepdims=True)
    probs2 = jnp.where(subl == i1, -jnp.inf, probsT)
    v2 = jnp.max(probs2, axis=0, keepdims=True)
    i2 = jnp.min(jnp.where(probs2 == v2, subl, E), axis=0, keepdims=True)

    oh0 = (subl == i1).astype(jnp.int32)
    oh1 = (subl == i2).astype(jnp.int32)
    inc0 = _lane_cumsum(oh0)
    inc1 = _lane_cumsum(oh1)
    exc0 = inc0 - oh0
    exc1 = inc1 - oh1
    tot0 = inc0[:, N - 1:N]
    tot1 = inc1[:, N - 1:N]
    counts = tot0 + tot1
    nb = (counts + (B - 1)) // B
    r8 = jax.lax.broadcasted_iota(jnp.int32, (E, E), 0)
    c8 = jax.lax.broadcasted_iota(jnp.int32, (E, E), 1)
    L8 = (r8 > c8).astype(jnp.float32)
    bs = jax.lax.dot_general(L8, nb.astype(jnp.float32),
                             (((1,), (0,)), ((), ())),
                             preferred_element_type=jnp.float32).astype(
                                 jnp.int32)
    pstart = B * bs
    slot0 = jnp.sum(oh0 * (pstart + exc0), axis=0, keepdims=True)
    slot1 = jnp.sum(oh1 * (pstart + tot0 + exc1), axis=0, keepdims=True)
    slots_ref[...] = jnp.concatenate([slot0, slot1], axis=0)
    gates_ref[...] = jnp.concatenate([v1, v2], axis=0)

    nb_total = bs[E - 1:E, 0:1] + nb[E - 1:E, 0:1]
    biota = jax.lax.broadcasted_iota(jnp.int32, (E, 32), 1)
    be = jnp.sum((biota >= bs).astype(jnp.int32), axis=0, keepdims=True) - 1
    e8 = jax.lax.broadcasted_iota(jnp.int32, (E, 1), 0)
    last_e = jnp.max(jnp.where(counts > 0, e8, 0), axis=0, keepdims=True)
    be = jnp.minimum(be, last_e)
    lane32 = jax.lax.broadcasted_iota(jnp.int32, (1, 32), 1)
    meta_ref[...] = jnp.where(lane32 == NBMAX, nb_total, be)


def _routing(x, gW1, gb1, gW2, gb2):
    return pl.pallas_call(
        _routing_kernel,
        in_specs=[
            pl.BlockSpec((N, D), lambda: (0, 0)),
            pl.BlockSpec((D, E), lambda: (0, 0)),
            pl.BlockSpec((E,), lambda: (0,)),
            pl.BlockSpec((E, E), lambda: (0, 0)),
            pl.BlockSpec((E,), lambda: (0,)),
        ],
        out_specs=[
            pl.BlockSpec((K, N), lambda: (0, 0)),
            pl.BlockSpec((K, N), lambda: (0, 0)),
            pl.BlockSpec((1, 32), lambda: (0, 0)),
        ],
        out_shape=[
            jax.ShapeDtypeStruct((K, N), jnp.int32),
            jax.ShapeDtypeStruct((K, N), jnp.float32),
            jax.ShapeDtypeStruct((1, 32), jnp.int32),
        ],
    )(x, gW1, gb1, gW2, gb2)


@functools.partial(
    pl.kernel, mesh=_mesh, compiler_params=_cp,
    out_type=[jax.ShapeDtypeStruct((P, 128), jnp.float32),
              jax.ShapeDtypeStruct((P, D), jnp.float32)],
    scratch_types=[
        pltpu.VMEM((4, 32), jnp.int32),
        pltpu.VMEM((PAIRS_W,), jnp.float32),
        pltpu.VMEM((PAIRS_W, 128), jnp.float32),
        pltpu.VMEM((32, D), jnp.float32),
        pltpu.VMEM((32, D), jnp.float32),
        pltpu.SemaphoreType.DMA,
        pltpu.SemaphoreType.DMA,
    ],
)
def _dispatch_kernel(x_hbm, slots_hbm, gates_hbm, grows_hbm, xs_hbm,
                     idx_v, gv, rows_v, xb0, xb1, sem, sem2):
    wid = lax.axis_index("s") * NC + lax.axis_index("c")
    k = wid // (N // PAIRS_W)
    n0 = (wid * PAIRS_W) % N

    for j in range(4):
        pltpu.sync_copy(slots_hbm.at[k, pl.ds(n0 + j * 32, 32)], idx_v.at[j])
    pltpu.sync_copy(gates_hbm.at[k, pl.ds(n0, PAIRS_W)], gv)

    col0 = jnp.zeros((16,), jnp.int32)
    for c in range(PAIRS_W // 16):
        ridx = lax.iota(jnp.int32, 16) + c * 16
        plsc.store_scatter(rows_v, [ridx, col0], gv[pl.ds(c * 16, 16)])
    cg = []
    for j in range(4):
        cg.append(pltpu.async_copy(
            rows_v.at[pl.ds(j * 32, 32)], grows_hbm.at[idx_v.at[j]], sem2))

    bufs = (xb0, xb1)
    cps = [None, None]
    for j in range(4):
        buf = bufs[j % 2]
        if cps[j % 2] is not None:
            cps[j % 2].wait()
        pltpu.sync_copy(x_hbm.at[pl.ds(n0 + j * 32, 32)], buf)
        cps[j % 2] = pltpu.async_copy(buf, xs_hbm.at[idx_v.at[j]], sem)
    for cp in cps:
        cp.wait()
    for cp in cg:
        cp.wait()


CHUNKS = 4
CW = H // CHUNKS


def _gemm_kernel(meta_ref, xs_ref, gate_ref, eW1_ref, eb1_ref, eW2_ref,
                 eb2_ref, ys_ref):
    b = pl.program_id(0)
    e = meta_ref[0, b]

    @pl.when(b < meta_ref[0, NBMAX])
    def _():
        subl = jax.lax.broadcasted_iota(jnp.int32, (E, H), 0)
        b1 = jnp.sum(jnp.where(subl == e, eb1_ref[...], 0.0), axis=0,
                     keepdims=True)
        b2 = jnp.sum(jnp.where(subl == e, eb2_ref[...], 0.0), axis=0,
                     keepdims=True)
        xb = xs_ref[...].astype(jnp.bfloat16)
        hs = []
        for c in range(CHUNKS):
            w1c = eW1_ref[0][:, c * CW:(c + 1) * CW].astype(jnp.bfloat16)
            hc = jax.lax.dot_general(xb, w1c, (((1,), (0,)), ((), ())),
                                     preferred_element_type=jnp.float32)
            hs.append(jnp.tanh(hc + b1[:, c * CW:(c + 1) * CW])
                      .astype(jnp.bfloat16))
        hb = jnp.concatenate(hs, axis=1)
        g = gate_ref[...][:, 0:1]
        for c in range(CHUNKS):
            w2c = eW2_ref[0][:, c * CW:(c + 1) * CW].astype(jnp.bfloat16)
            yc = jax.lax.dot_general(hb, w2c, (((1,), (0,)), ((), ())),
                                     preferred_element_type=jnp.float32)
            ys_ref[:, c * CW:(c + 1) * CW] = \
                (yc + b2[:, c * CW:(c + 1) * CW]) * g


def _gemm(meta, xs, gate_rows, eW1, eb1, eW2, eb2):
    grid_spec = pltpu.PrefetchScalarGridSpec(
        num_scalar_prefetch=1,
        grid=(NBMAX,),
        in_specs=[
            pl.BlockSpec((B, D),
                         lambda b, m: (jnp.minimum(b, m[0, NBMAX] - 1), 0)),
            pl.BlockSpec((B, 128),
                         lambda b, m: (jnp.minimum(b, m[0, NBMAX] - 1), 0)),
            pl.BlockSpec((1, D, H), lambda b, m: (m[0, b], 0, 0)),
            pl.BlockSpec((E, H), lambda b, m: (0, 0)),
            pl.BlockSpec((1, H, D), lambda b, m: (m[0, b], 0, 0)),
            pl.BlockSpec((E, D), lambda b, m: (0, 0)),
        ],
        out_specs=pl.BlockSpec(
            (B, D), lambda b, m: (jnp.minimum(b, m[0, NBMAX] - 1), 0)),
    )
    return pl.pallas_call(
        _gemm_kernel,
        grid_spec=grid_spec,
        out_shape=jax.ShapeDtypeStruct((P, D), jnp.float32),
        compiler_params=pltpu.CompilerParams(
            dimension_semantics=("arbitrary",),
        ),
    )(meta, xs, gate_rows, eW1, eb1, eW2, eb2)


@functools.partial(
    pl.kernel, mesh=_mesh, compiler_params=_cp,
    out_type=jax.ShapeDtypeStruct((N, D), jnp.float32),
    scratch_types=[
        pltpu.VMEM((4, 16), jnp.int32),
        pltpu.VMEM((4, 16), jnp.int32),
        pltpu.VMEM((16, D), jnp.float32),
        pltpu.VMEM((16, D), jnp.float32),
        pltpu.VMEM((16, D), jnp.float32),
        pltpu.VMEM((16, D), jnp.float32),
        pltpu.SemaphoreType.DMA,
        pltpu.SemaphoreType.DMA,
        pltpu.SemaphoreType.DMA,
    ],
)
def _combine_kernel(ys_hbm, slots_hbm, out_hbm, idx0, idx1,
                    a0, a1, b0, b1, sem, semb, semo):
    wid = lax.axis_index("s") * NC + lax.axis_index("c")
    n0 = wid * TOK_W
    for j in range(4):
        pltpu.sync_copy(slots_hbm.at[0, pl.ds(n0 + j * 16, 16)], idx0.at[j])
        pltpu.sync_copy(slots_hbm.at[1, pl.ds(n0 + j * 16, 16)], idx1.at[j])

    pairs = ((a0, a1, sem), (b0, b1, semb))
    gets = [None, None]
    outs = [None, None]

    def issue(j):
        p0, p1, s = pairs[j % 2]
        c0 = pltpu.async_copy(ys_hbm.at[idx0.at[j]], p0, s)
        c1 = pltpu.async_copy(ys_hbm.at[idx1.at[j]], p1, s)
        gets[j % 2] = (c0, c1)

    issue(0)
    for j in range(4):
        p0, p1, _ = pairs[j % 2]
        c0, c1 = gets[j % 2]
        c0.wait()
        c1.wait()
        if j + 1 < 4:
            if outs[(j + 1) % 2] is not None:
                outs[(j + 1) % 2].wait()
                outs[(j + 1) % 2] = None
            issue(j + 1)

        @pl.loop(0, 16)
        def _(r):
            for q in range(D // 16):
                sl = pl.ds(q * 16, 16)
                plsc.addupdate(p0.at[r, sl], p1[r, sl])

        outs[j % 2] = pltpu.async_copy(
            p0, out_hbm.at[pl.ds(n0 + j * 16, 16)], semo)
    for o in outs:
        if o is not None:
            o.wait()


@jax.jit
def _moe(x, gW1, gb1, gW2, gb2, eW1, eb1, eW2, eb2):
    slots, gates, meta = _routing(x, gW1, gb1, gW2, gb2)
    gate_rows, xs = _dispatch_kernel(x, slots, gates)
    ys = _gemm(meta, xs, gate_rows, eW1, eb1, eW2, eb2)
    return _combine_kernel(ys, slots)


def kernel(x, gW1, gb1, gW2, gb2, eW1, eb1, eW2, eb2, train):
    del train
    return _moe(x, gW1, gb1, gW2, gb2, eW1, eb1, eW2, eb2)


# speedup vs baseline: 1.2584x; 1.0354x over previous
import dataclasses
import functools

import jax
import jax.numpy as jnp
from jax import lax
from jax.experimental import pallas as pl
from jax.experimental.pallas import tpu as pltpu
from jax.experimental.pallas import tpu_sc as plsc

N, D, H, E, K = 2048, 1024, 1024, 8, 2
B = 512
NBMAX = N * K // B + E - 1
P = NBMAX * B
NC, NS = 2, 16
NW = NC * NS
PAIRS_W = N * K // NW
TOK_W = N // NW

_mesh = plsc.VectorSubcoreMesh(core_axis_name="c", subcore_axis_name="s")
_cp = pltpu.CompilerParams()
if "needs_layout_passes" in pltpu.CompilerParams.__dataclass_fields__:
    _cp = dataclasses.replace(_cp, needs_layout_passes=False)


def _lane_cumsum(x):
    r, c = x.shape
    lane = jax.lax.broadcasted_iota(jnp.int32, (r, c), 1)
    acc = x
    sh = 1
    while sh < c:
        rolled = pltpu.roll(acc, sh, axis=1)
        acc = acc + jnp.where(lane >= sh, rolled, 0)
        sh *= 2
    return acc


def _routing_kernel(x_ref, gW1_ref, gb1_ref, gW2_ref, gb2_ref,
                    slots_ref, gates_ref, meta_ref):
    x = x_ref[...]
    l1 = jnp.tanh(
        jax.lax.dot_general(x, gW1_ref[...], (((1,), (0,)), ((), ())),
                            preferred_element_type=jnp.float32)
        + gb1_ref[...].reshape(1, E))
    logits = jax.lax.dot_general(l1, gW2_ref[...], (((1,), (0,)), ((), ())),
                                 preferred_element_type=jnp.float32) \
        + gb2_ref[...].reshape(1, E)
    mx = jnp.max(logits, axis=-1, keepdims=True)
    exn = jnp.exp(logits - mx)
    probs = exn / jnp.sum(exn, axis=-1, keepdims=True)
    probsT = jnp.transpose(probs)

    subl = jax.lax.broadcasted_iota(jnp.int32, (E, N), 0)
    v1 = jnp.max(probsT, axis=0, keepdims=True)
    i1 = jnp.min(jnp.where(probsT == v1, subl, E), axis=0, keepdims=True)
    probs2 = jnp.where(subl == i1, -jnp.inf, probsT)
    v2 = jnp.max(probs2, axis=0, keepdims=True)
    i2 = jnp.min(jnp.where(probs2 == v2, subl, E), axis=0, keepdims=True)

    oh0 = (subl == i1).astype(jnp.int32)
    oh1 = (subl == i2).astype(jnp.int32)
    inc0 = _lane_cumsum(oh0)
    inc1 = _lane_cumsum(oh1)
    exc0 = inc0 - oh0
    exc1 = inc1 - oh1
    tot0 = inc0[:, N - 1:N]
    tot1 = inc1[:, N - 1:N]
    counts = tot0 + tot1
    nb = (counts + (B - 1)) // B
    r8 = jax.lax.broadcasted_iota(jnp.int32, (E, E), 0)
    c8 = jax.lax.broadcasted_iota(jnp.int32, (E, E), 1)
    L8 = (r8 > c8).astype(jnp.float32)
    bs = jax.lax.dot_general(L8, nb.astype(jnp.float32),
                             (((1,), (0,)), ((), ())),
                             preferred_element_type=jnp.float32).astype(
                                 jnp.int32)
    pstart = B * bs
    slot0 = jnp.sum(oh0 * (pstart + exc0), axis=0, keepdims=True)
    slot1 = jnp.sum(oh1 * (pstart + tot0 + exc1), axis=0, keepdims=True)
    slots_ref[...] = jnp.concatenate([slot0, slot1], axis=0)
    gates_ref[...] = jnp.concatenate([v1, v2], axis=0)

    nb_total = bs[E - 1:E, 0:1] + nb[E - 1:E, 0:1]
    biota = jax.lax.broadcasted_iota(jnp.int32, (E, 32), 1)
    be = jnp.sum((biota >= bs).astype(jnp.int32), axis=0, keepdims=True) - 1
    e8 = jax.lax.broadcasted_iota(jnp.int32, (E, 1), 0)
    last_e = jnp.max(jnp.where(counts > 0, e8, 0), axis=0, keepdims=True)
    be = jnp.minimum(be, last_e)
    lane32 = jax.lax.broadcasted_iota(jnp.int32, (1, 32), 1)
    meta_ref[...] = jnp.where(lane32 == NBMAX, nb_total, be)


def _routing(x, gW1, gb1, gW2, gb2):
    return pl.pallas_call(
        _routing_kernel,
        in_specs=[
            pl.BlockSpec((N, D), lambda: (0, 0)),
            pl.BlockSpec((D, E), lambda: (0, 0)),
            pl.BlockSpec((E,), lambda: (0,)),
            pl.BlockSpec((E, E), lambda: (0, 0)),
            pl.BlockSpec((E,), lambda: (0,)),
        ],
        out_specs=[
            pl.BlockSpec((K, N), lambda: (0, 0)),
            pl.BlockSpec((K, N), lambda: (0, 0)),
            pl.BlockSpec((1, 32), lambda: (0, 0)),
        ],
        out_shape=[
            jax.ShapeDtypeStruct((K, N), jnp.int32),
            jax.ShapeDtypeStruct((K, N), jnp.float32),
            jax.ShapeDtypeStruct((1, 32), jnp.int32),
        ],
    )(x, gW1, gb1, gW2, gb2)


@functools.partial(
    pl.kernel, mesh=_mesh, compiler_params=_cp,
    out_type=[jax.ShapeDtypeStruct((P, 128), jnp.float32),
              jax.ShapeDtypeStruct((P, D), jnp.float32)],
    scratch_types=[
        pltpu.VMEM((4, 32), jnp.int32),
        pltpu.VMEM((PAIRS_W,), jnp.float32),
        pltpu.VMEM((PAIRS_W, 128), jnp.float32),
        pltpu.VMEM((32, D), jnp.float32),
        pltpu.VMEM((32, D), jnp.float32),
        pltpu.SemaphoreType.DMA,
        pltpu.SemaphoreType.DMA,
    ],
)
def _dispatch_kernel(x_hbm, slots_hbm, gates_hbm, grows_hbm, xs_hbm,
                     idx_v, gv, rows_v, xb0, xb1, sem, sem2):
    wid = lax.axis_index("s") * NC + lax.axis_index("c")
    k = wid // (N // PAIRS_W)
    n0 = (wid * PAIRS_W) % N

    pre = [pltpu.async_copy(slots_hbm.at[k, pl.ds(n0 + j * 32, 32)],
                            idx_v.at[j], sem2) for j in range(4)]
    pre.append(pltpu.async_copy(gates_hbm.at[k, pl.ds(n0, PAIRS_W)], gv, sem2))
    for cp in pre:
        cp.wait()

    col0 = jnp.zeros((16,), jnp.int32)
    for c in range(PAIRS_W // 16):
        ridx = lax.iota(jnp.int32, 16) + c * 16
        plsc.store_scatter(rows_v, [ridx, col0], gv[pl.ds(c * 16, 16)])
    cg = []
    for j in range(4):
        cg.append(pltpu.async_copy(
            rows_v.at[pl.ds(j * 32, 32)], grows_hbm.at[idx_v.at[j]], sem2))

    bufs = (xb0, xb1)
    cps = [None, None]
    for j in range(4):
        buf = bufs[j % 2]
        if cps[j % 2] is not None:
            cps[j % 2].wait()
        pltpu.sync_copy(x_hbm.at[pl.ds(n0 + j * 32, 32)], buf)
        cps[j % 2] = pltpu.async_copy(buf, xs_hbm.at[idx_v.at[j]], sem)
    for cp in cps:
        cp.wait()
    for cp in cg:
        cp.wait()


CHUNKS = 4
CW = H // CHUNKS


def _gemm_kernel(meta_ref, xs_ref, gate_ref, eW1_ref, eb1_ref, eW2_ref,
                 eb2_ref, ys_ref):
    b = pl.program_id(0)
    e = meta_ref[0, b]

    @pl.when(b < meta_ref[0, NBMAX])
    def _():
        subl = jax.lax.broadcasted_iota(jnp.int32, (E, H), 0)
        b1 = jnp.sum(jnp.where(subl == e, eb1_ref[...], 0.0), axis=0,
                     keepdims=True)
        b2 = jnp.sum(jnp.where(subl == e, eb2_ref[...], 0.0), axis=0,
                     keepdims=True)
        xb = xs_ref[...].astype(jnp.bfloat16)
        hs = []
        for c in range(CHUNKS):
            w1c = eW1_ref[0][:, c * CW:(c + 1) * CW].astype(jnp.bfloat16)
            hc = jax.lax.dot_general(xb, w1c, (((1,), (0,)), ((), ())),
                                     preferred_element_type=jnp.float32)
            hs.append(jnp.tanh(hc + b1[:, c * CW:(c + 1) * CW])
                      .astype(jnp.bfloat16))
        hb = jnp.concatenate(hs, axis=1)
        g = gate_ref[...][:, 0:1]
        for c in range(CHUNKS):
            w2c = eW2_ref[0][:, c * CW:(c + 1) * CW].astype(jnp.bfloat16)
            yc = jax.lax.dot_general(hb, w2c, (((1,), (0,)), ((), ())),
                                     preferred_element_type=jnp.float32)
            ys_ref[:, c * CW:(c + 1) * CW] = \
                (yc + b2[:, c * CW:(c + 1) * CW]) * g


def _gemm(meta, xs, gate_rows, eW1, eb1, eW2, eb2):
    grid_spec = pltpu.PrefetchScalarGridSpec(
        num_scalar_prefetch=1,
        grid=(NBMAX,),
        in_specs=[
            pl.BlockSpec((B, D),
                         lambda b, m: (jnp.minimum(b, m[0, NBMAX] - 1), 0)),
            pl.BlockSpec((B, 128),
                         lambda b, m: (jnp.minimum(b, m[0, NBMAX] - 1), 0)),
            pl.BlockSpec((1, D, H), lambda b, m: (m[0, b], 0, 0)),
            pl.BlockSpec((E, H), lambda b, m: (0, 0)),
            pl.BlockSpec((1, H, D), lambda b, m: (m[0, b], 0, 0)),
            pl.BlockSpec((E, D), lambda b, m: (0, 0)),
        ],
        out_specs=pl.BlockSpec(
            (B, D), lambda b, m: (jnp.minimum(b, m[0, NBMAX] - 1), 0)),
    )
    return pl.pallas_call(
        _gemm_kernel,
        grid_spec=grid_spec,
        out_shape=jax.ShapeDtypeStruct((P, D), jnp.float32),
        compiler_params=pltpu.CompilerParams(
            dimension_semantics=("arbitrary",),
        ),
    )(meta, xs, gate_rows, eW1, eb1, eW2, eb2)


@functools.partial(
    pl.kernel, mesh=_mesh, compiler_params=_cp,
    out_type=jax.ShapeDtypeStruct((N, D), jnp.float32),
    scratch_types=[
        pltpu.VMEM((4, 16), jnp.int32),
        pltpu.VMEM((4, 16), jnp.int32),
        pltpu.VMEM((16, D), jnp.float32),
        pltpu.VMEM((16, D), jnp.float32),
        pltpu.VMEM((16, D), jnp.float32),
        pltpu.VMEM((16, D), jnp.float32),
        pltpu.SemaphoreType.DMA,
        pltpu.SemaphoreType.DMA,
        pltpu.SemaphoreType.DMA,
    ],
)
def _combine_kernel(ys_hbm, slots_hbm, out_hbm, idx0, idx1,
                    a0, a1, b0, b1, sem, semb, semo):
    wid = lax.axis_index("s") * NC + lax.axis_index("c")
    n0 = wid * TOK_W
    pre = []
    for j in range(4):
        pre.append(pltpu.async_copy(slots_hbm.at[0, pl.ds(n0 + j * 16, 16)],
                                    idx0.at[j], semo))
        pre.append(pltpu.async_copy(slots_hbm.at[1, pl.ds(n0 + j * 16, 16)],
                                    idx1.at[j], semo))
    for cp in pre:
        cp.wait()

    pairs = ((a0, a1, sem), (b0, b1, semb))
    gets = [None, None]
    outs = [None, None]

    def issue(j):
        p0, p1, s = pairs[j % 2]
        c0 = pltpu.async_copy(ys_hbm.at[idx0.at[j]], p0, s)
        c1 = pltpu.async_copy(ys_hbm.at[idx1.at[j]], p1, s)
        gets[j % 2] = (c0, c1)

    issue(0)
    for j in range(4):
        p0, p1, _ = pairs[j % 2]
        c0, c1 = gets[j % 2]
        c0.wait()
        c1.wait()
        if j + 1 < 4:
            if outs[(j + 1) % 2] is not None:
                outs[(j + 1) % 2].wait()
                outs[(j + 1) % 2] = None
            issue(j + 1)

        @pl.loop(0, 16)
        def _(r):
            for q in range(D // 16):
                sl = pl.ds(q * 16, 16)
                plsc.addupdate(p0.at[r, sl], p1[r, sl])

        outs[j % 2] = pltpu.async_copy(
            p0, out_hbm.at[pl.ds(n0 + j * 16, 16)], semo)
    for o in outs:
        if o is not None:
            o.wait()


@jax.jit
def _moe(x, gW1, gb1, gW2, gb2, eW1, eb1, eW2, eb2):
    slots, gates, meta = _routing(x, gW1, gb1, gW2, gb2)
    gate_rows, xs = _dispatch_kernel(x, slots, gates)
    ys = _gemm(meta, xs, gate_rows, eW1, eb1, eW2, eb2)
    return _combine_kernel(ys, slots)


def kernel(x, gW1, gb1, gW2, gb2, eW1, eb1, eW2, eb2, train):
    del train
    return _moe(x, gW1, gb1, gW2, gb2, eW1, eb1, eW2, eb2)


# speedup vs baseline: 1.2918x; 1.0266x over previous
import dataclasses
import functools

import jax
import jax.numpy as jnp
from jax import lax
from jax.experimental import pallas as pl
from jax.experimental.pallas import tpu as pltpu
from jax.experimental.pallas import tpu_sc as plsc

N, D, H, E, K = 2048, 1024, 1024, 8, 2
B = 512
NBMAX = N * K // B + E - 1
P = NBMAX * B
NC, NS = 2, 16
NW = NC * NS
PAIRS_W = N * K // NW
TOK_W = N // NW

_mesh = plsc.VectorSubcoreMesh(core_axis_name="c", subcore_axis_name="s")
_cp = pltpu.CompilerParams()
if "needs_layout_passes" in pltpu.CompilerParams.__dataclass_fields__:
    _cp = dataclasses.replace(_cp, needs_layout_passes=False)


def _lane_cumsum(x):
    r, c = x.shape
    lane = jax.lax.broadcasted_iota(jnp.int32, (r, c), 1)
    acc = x
    sh = 1
    while sh < c:
        rolled = pltpu.roll(acc, sh, axis=1)
        acc = acc + jnp.where(lane >= sh, rolled, 0)
        sh *= 2
    return acc


def _routing_kernel(x_ref, gW1_ref, gb1_ref, gW2_ref, gb2_ref,
                    slots_ref, gates_ref, meta_ref):
    x = x_ref[...]
    l1 = jnp.tanh(
        jax.lax.dot_general(x, gW1_ref[...], (((1,), (0,)), ((), ())),
                            preferred_element_type=jnp.float32)
        + gb1_ref[...].reshape(1, E))
    logits = jax.lax.dot_general(l1, gW2_ref[...], (((1,), (0,)), ((), ())),
                                 preferred_element_type=jnp.float32) \
        + gb2_ref[...].reshape(1, E)
    mx = jnp.max(logits, axis=-1, keepdims=True)
    exn = jnp.exp(logits - mx)
    probs = exn / jnp.sum(exn, axis=-1, keepdims=True)
    probsT = jnp.transpose(probs)

    subl = jax.lax.broadcasted_iota(jnp.int32, (E, N), 0)
    v1 = jnp.max(probsT, axis=0, keepdims=True)
    i1 = jnp.min(jnp.where(probsT == v1, subl, E), axis=0, keepdims=True)
    probs2 = jnp.where(subl == i1, -jnp.inf, probsT)
    v2 = jnp.max(probs2, axis=0, keepdims=True)
    i2 = jnp.min(jnp.where(probs2 == v2, subl, E), axis=0, keepdims=True)

    oh0 = (subl == i1).astype(jnp.int32)
    oh1 = (subl == i2).astype(jnp.int32)
    inc0 = _lane_cumsum(oh0)
    inc1 = _lane_cumsum(oh1)
    exc0 = inc0 - oh0
    exc1 = inc1 - oh1
    tot0 = inc0[:, N - 1:N]
    tot1 = inc1[:, N - 1:N]
    counts = tot0 + tot1
    nb = (counts + (B - 1)) // B
    r8 = jax.lax.broadcasted_iota(jnp.int32, (E, E), 0)
    c8 = jax.lax.broadcasted_iota(jnp.int32, (E, E), 1)
    L8 = (r8 > c8).astype(jnp.float32)
    bs = jax.lax.dot_general(L8, nb.astype(jnp.float32),
                             (((1,), (0,)), ((), ())),
                             preferred_element_type=jnp.float32).astype(
                                 jnp.int32)
    pstart = B * bs
    slot0 = jnp.sum(oh0 * (pstart + exc0), axis=0, keepdims=True)
    slot1 = jnp.sum(oh1 * (pstart + tot0 + exc1), axis=0, keepdims=True)
    slots_ref[...] = jnp.concatenate([slot0, slot1], axis=0)
    gates_ref[...] = jnp.concatenate([v1, v2], axis=0)

    nb_total = bs[E - 1:E, 0:1] + nb[E - 1:E, 0:1]
    biota = jax.lax.broadcasted_iota(jnp.int32, (E, 32), 1)
    be = jnp.sum((biota >= bs).astype(jnp.int32), axis=0, keepdims=True) - 1
    e8 = jax.lax.broadcasted_iota(jnp.int32, (E, 1), 0)
    last_e = jnp.max(jnp.where(counts > 0, e8, 0), axis=0, keepdims=True)
    be = jnp.minimum(be, last_e)
    lane32 = jax.lax.broadcasted_iota(jnp.int32, (1, 32), 1)
    meta_ref[...] = jnp.where(lane32 == NBMAX, nb_total, be)


def _routing(x, gW1, gb1, gW2, gb2):
    return pl.pallas_call(
        _routing_kernel,
        in_specs=[
            pl.BlockSpec((N, D), lambda: (0, 0)),
            pl.BlockSpec((D, E), lambda: (0, 0)),
            pl.BlockSpec((E,), lambda: (0,)),
            pl.BlockSpec((E, E), lambda: (0, 0)),
            pl.BlockSpec((E,), lambda: (0,)),
        ],
        out_specs=[
            pl.BlockSpec((K, N), lambda: (0, 0)),
            pl.BlockSpec((K, N), lambda: (0, 0)),
            pl.BlockSpec((1, 32), lambda: (0, 0)),
        ],
        out_shape=[
            jax.ShapeDtypeStruct((K, N), jnp.int32),
            jax.ShapeDtypeStruct((K, N), jnp.float32),
            jax.ShapeDtypeStruct((1, 32), jnp.int32),
        ],
    )(x, gW1, gb1, gW2, gb2)


@functools.partial(
    pl.kernel, mesh=_mesh, compiler_params=_cp,
    out_type=[jax.ShapeDtypeStruct((P, 128), jnp.float32),
              jax.ShapeDtypeStruct((P, D), jnp.float32)],
    scratch_types=[
        pltpu.VMEM((4, 32), jnp.int32),
        pltpu.VMEM((PAIRS_W,), jnp.float32),
        pltpu.VMEM((PAIRS_W, 128), jnp.float32),
        pltpu.VMEM((32, D), jnp.float32),
        pltpu.VMEM((32, D), jnp.float32),
        pltpu.SemaphoreType.DMA,
        pltpu.SemaphoreType.DMA,
    ],
)
def _dispatch_kernel(x_hbm, slots_hbm, gates_hbm, grows_hbm, xs_hbm,
                     idx_v, gv, rows_v, xb0, xb1, sem, sem2):
    wid = lax.axis_index("s") * NC + lax.axis_index("c")
    k = wid // (N // PAIRS_W)
    n0 = (wid * PAIRS_W) % N

    pre = [pltpu.async_copy(slots_hbm.at[k, pl.ds(n0 + j * 32, 32)],
                            idx_v.at[j], sem2) for j in range(4)]
    pre.append(pltpu.async_copy(gates_hbm.at[k, pl.ds(n0, PAIRS_W)], gv, sem2))
    for cp in pre:
        cp.wait()

    bufs = (xb0, xb1)
    lds = [None, None]
    scs = [None, None]
    lds[0] = pltpu.async_copy(x_hbm.at[pl.ds(n0, 32)], xb0, sem)

    col0 = jnp.zeros((16,), jnp.int32)
    for c in range(PAIRS_W // 16):
        ridx = lax.iota(jnp.int32, 16) + c * 16
        plsc.store_scatter(rows_v, [ridx, col0], gv[pl.ds(c * 16, 16)])
    cg = []
    for j in range(4):
        cg.append(pltpu.async_copy(
            rows_v.at[pl.ds(j * 32, 32)], grows_hbm.at[idx_v.at[j]], sem2))

    for j in range(4):
        if j + 1 < 4:
            nb = (j + 1) % 2
            if scs[nb] is not None:
                scs[nb].wait()
            lds[nb] = pltpu.async_copy(
                x_hbm.at[pl.ds(n0 + (j + 1) * 32, 32)], bufs[nb], sem)
        lds[j % 2].wait()
        scs[j % 2] = pltpu.async_copy(bufs[j % 2], xs_hbm.at[idx_v.at[j]],
                                      sem2)
    for cp in scs:
        cp.wait()
    for cp in cg:
        cp.wait()


CHUNKS = 4
CW = H // CHUNKS


def _gemm_kernel(meta_ref, xs_ref, gate_ref, eW1_ref, eb1_ref, eW2_ref,
                 eb2_ref, ys_ref):
    b = pl.program_id(0)
    e = meta_ref[0, b]

    @pl.when(b < meta_ref[0, NBMAX])
    def _():
        subl = jax.lax.broadcasted_iota(jnp.int32, (E, H), 0)
        b1 = jnp.sum(jnp.where(subl == e, eb1_ref[...], 0.0), axis=0,
                     keepdims=True)
        b2 = jnp.sum(jnp.where(subl == e, eb2_ref[...], 0.0), axis=0,
                     keepdims=True)
        xb = xs_ref[...].astype(jnp.bfloat16)
        hs = []
        for c in range(CHUNKS):
            w1c = eW1_ref[0][:, c * CW:(c + 1) * CW].astype(jnp.bfloat16)
            hc = jax.lax.dot_general(xb, w1c, (((1,), (0,)), ((), ())),
                                     preferred_element_type=jnp.float32)
            hs.append(jnp.tanh(hc + b1[:, c * CW:(c + 1) * CW])
                      .astype(jnp.bfloat16))
        hb = jnp.concatenate(hs, axis=1)
        g = gate_ref[...][:, 0:1]
        for c in range(CHUNKS):
            w2c = eW2_ref[0][:, c * CW:(c + 1) * CW].astype(jnp.bfloat16)
            yc = jax.lax.dot_general(hb, w2c, (((1,), (0,)), ((), ())),
                                     preferred_element_type=jnp.float32)
            ys_ref[:, c * CW:(c + 1) * CW] = \
                (yc + b2[:, c * CW:(c + 1) * CW]) * g


def _gemm(meta, xs, gate_rows, eW1, eb1, eW2, eb2):
    grid_spec = pltpu.PrefetchScalarGridSpec(
        num_scalar_prefetch=1,
        grid=(NBMAX,),
        in_specs=[
            pl.BlockSpec((B, D),
                         lambda b, m: (jnp.minimum(b, m[0, NBMAX] - 1), 0)),
            pl.BlockSpec((B, 128),
                         lambda b, m: (jnp.minimum(b, m[0, NBMAX] - 1), 0)),
            pl.BlockSpec((1, D, H), lambda b, m: (m[0, b], 0, 0)),
            pl.BlockSpec((E, H), lambda b, m: (0, 0)),
            pl.BlockSpec((1, H, D), lambda b, m: (m[0, b], 0, 0)),
            pl.BlockSpec((E, D), lambda b, m: (0, 0)),
        ],
        out_specs=pl.BlockSpec(
            (B, D), lambda b, m: (jnp.minimum(b, m[0, NBMAX] - 1), 0)),
    )
    return pl.pallas_call(
        _gemm_kernel,
        grid_spec=grid_spec,
        out_shape=jax.ShapeDtypeStruct((P, D), jnp.float32),
        compiler_params=pltpu.CompilerParams(
            dimension_semantics=("arbitrary",),
        ),
    )(meta, xs, gate_rows, eW1, eb1, eW2, eb2)


@functools.partial(
    pl.kernel, mesh=_mesh, compiler_params=_cp,
    out_type=jax.ShapeDtypeStruct((N, D), jnp.float32),
    scratch_types=[
        pltpu.VMEM((4, 16), jnp.int32),
        pltpu.VMEM((4, 16), jnp.int32),
        pltpu.VMEM((16, D), jnp.float32),
        pltpu.VMEM((16, D), jnp.float32),
        pltpu.VMEM((16, D), jnp.float32),
        pltpu.VMEM((16, D), jnp.float32),
        pltpu.SemaphoreType.DMA,
        pltpu.SemaphoreType.DMA,
        pltpu.SemaphoreType.DMA,
    ],
)
def _combine_kernel(ys_hbm, slots_hbm, out_hbm, idx0, idx1,
                    a0, a1, b0, b1, sem, semb, semo):
    wid = lax.axis_index("s") * NC + lax.axis_index("c")
    n0 = wid * TOK_W
    pre = []
    for j in range(4):
        pre.append(pltpu.async_copy(slots_hbm.at[0, pl.ds(n0 + j * 16, 16)],
                                    idx0.at[j], semo))
        pre.append(pltpu.async_copy(slots_hbm.at[1, pl.ds(n0 + j * 16, 16)],
                                    idx1.at[j], semo))
    for cp in pre:
        cp.wait()

    pairs = ((a0, a1, sem), (b0, b1, semb))
    gets = [None, None]
    outs = [None, None]

    def issue(j):
        p0, p1, s = pairs[j % 2]
        c0 = pltpu.async_copy(ys_hbm.at[idx0.at[j]], p0, s)
        c1 = pltpu.async_copy(ys_hbm.at[idx1.at[j]], p1, s)
        gets[j % 2] = (c0, c1)

    issue(0)
    for j in range(4):
        p0, p1, _ = pairs[j % 2]
        c0, c1 = gets[j % 2]
        c0.wait()
        c1.wait()
        if j + 1 < 4:
            if outs[(j + 1) % 2] is not None:
                outs[(j + 1) % 2].wait()
                outs[(j + 1) % 2] = None
            issue(j + 1)

        @pl.loop(0, 16)
        def _(r):
            for q in range(D // 16):
                sl = pl.ds(q * 16, 16)
                plsc.addupdate(p0.at[r, sl], p1[r, sl])

        outs[j % 2] = pltpu.async_copy(
            p0, out_hbm.at[pl.ds(n0 + j * 16, 16)], semo)
    for o in outs:
        if o is not None:
            o.wait()


@jax.jit
def _moe(x, gW1, gb1, gW2, gb2, eW1, eb1, eW2, eb2):
    slots, gates, meta = _routing(x, gW1, gb1, gW2, gb2)
    gate_rows, xs = _dispatch_kernel(x, slots, gates)
    ys = _gemm(meta, xs, gate_rows, eW1, eb1, eW2, eb2)
    return _combine_kernel(ys, slots)


def kernel(x, gW1, gb1, gW2, gb2, eW1, eb1, eW2, eb2, train):
    del train
    return _moe(x, gW1, gb1, gW2, gb2, eW1, eb1, eW2, eb2)
